# baseline (speedup 1.0000x reference)
import jax
import jax.numpy as jnp
from jax.experimental import pallas as pl
from jax.experimental.pallas import tpu as pltpu

_E = 64
_D = 1024
_N = 128
_G = 4
_DG = _D // _G


def _moe_body(ids_ref, nused_ref, m_ref, x_ref, *rest):
    w_refs = rest[:_G]
    b_ref, o_ref = rest[_G], rest[_G + 1]
    i = pl.program_id(0)

    @pl.when(i == 0)
    def _init():
        o_ref[...] = jnp.zeros_like(o_ref)

    @pl.when(i < nused_ref[0])
    def _step():
        e = ids_ref[i]
        mask = m_ref[...] == e
        x = x_ref[...].astype(jnp.bfloat16)
        for g in range(_G):
            xw = jax.lax.dot_general(
                x, w_refs[g][0].astype(jnp.bfloat16),
                dimension_numbers=(((1,), (1,)), ((), ())),
                preferred_element_type=jnp.float32,
            )
            contrib = xw + b_ref[0, :, g * _DG:(g + 1) * _DG]
            o_ref[:, g * _DG:(g + 1) * _DG] += jnp.where(mask, contrib, 0.0)


def kernel(x, module_indices, W, b):
    m = module_indices.astype(jnp.int32)
    s = jnp.sort(m)
    is_new = jnp.concatenate([jnp.ones((1,), jnp.bool_), s[1:] != s[:-1]])
    pos = jnp.cumsum(is_new.astype(jnp.int32)) - 1
    ids = jnp.full((_E,), s[-1], jnp.int32).at[pos].set(s)
    n_used = pos[-1:] + 1

    m2d = m.reshape(_N, 1)
    b3 = b.reshape(_E, 1, _D)

    def w_spec(g):
        return pl.BlockSpec(
            (1, _DG, _D), lambda i, ids, nu, g=g: (ids[i], g, 0))

    grid_spec = pltpu.PrefetchScalarGridSpec(
        num_scalar_prefetch=2,
        grid=(_E,),
        in_specs=[
            pl.BlockSpec((_N, 1), lambda i, ids, nu: (0, 0)),
            pl.BlockSpec((_N, _D), lambda i, ids, nu: (0, 0)),
            *[w_spec(g) for g in range(_G)],
            pl.BlockSpec((1, 1, _D), lambda i, ids, nu: (ids[i], 0, 0)),
        ],
        out_specs=pl.BlockSpec((_N, _D), lambda i, ids, nu: (0, 0)),
    )

    out = pl.pallas_call(
        _moe_body,
        grid_spec=grid_spec,
        out_shape=jax.ShapeDtypeStruct((_N, _D), jnp.float32),
        compiler_params=pltpu.CompilerParams(
            dimension_semantics=("arbitrary",),
        ),
    )(ids, n_used, m2d, x, *([W] * _G), b3)
    return out


# speedup vs baseline: 1.2011x; 1.2011x over previous
import jax
import jax.numpy as jnp
from jax.experimental import pallas as pl
from jax.experimental.pallas import tpu as pltpu

_E = 64
_D = 1024
_N = 128


def _moe_body(ids_ref, m_ref, x_ref, w0_ref, w1_ref, b_ref, o_ref):
    i = pl.program_id(0)

    @pl.when(i == 0)
    def _init():
        o_ref[...] = jnp.zeros_like(o_ref)

    e0 = ids_ref[2 * i]
    e1 = ids_ref[2 * i + 1]
    x = x_ref[...]
    xw0 = jax.lax.dot_general(
        x, w0_ref[0], dimension_numbers=(((1,), (1,)), ((), ())),
        preferred_element_type=jnp.float32,
    ) + b_ref[0, 0:1, :]
    xw1 = jax.lax.dot_general(
        x, w1_ref[0], dimension_numbers=(((1,), (1,)), ((), ())),
        preferred_element_type=jnp.float32,
    ) + b_ref[0, 1:2, :]
    m = m_ref[...]
    acc = jnp.where(m == e0, xw0, 0.0)
    acc = acc + jnp.where((m == e1) & (e1 != e0), xw1, 0.0)
    o_ref[...] += acc


def kernel(x, module_indices, W, b):
    m = module_indices.astype(jnp.int32)
    s = jnp.sort(m)
    is_new = jnp.concatenate([jnp.ones((1,), jnp.bool_), s[1:] != s[:-1]])
    pos = jnp.cumsum(is_new.astype(jnp.int32)) - 1
    ids = jnp.full((_E,), s[-1], jnp.int32).at[pos].set(s)
    n_used = pos[-1] + 1
    n_pairs = (n_used + 1) // 2

    m2d = m.reshape(_N, 1)

    grid_spec = pltpu.PrefetchScalarGridSpec(
        num_scalar_prefetch=1,
        grid=(n_pairs,),
        in_specs=[
            pl.BlockSpec((_N, 1), lambda i, ids: (0, 0)),
            pl.BlockSpec((_N, _D), lambda i, ids: (0, 0)),
            pl.BlockSpec((1, _D, _D), lambda i, ids: (ids[2 * i], 0, 0)),
            pl.BlockSpec((1, _D, _D), lambda i, ids: (ids[2 * i + 1], 0, 0)),
            pl.BlockSpec((1, 2, _D), lambda i, ids: (i, 0, 0)),
        ],
        out_specs=pl.BlockSpec((_N, _D), lambda i, ids: (0, 0)),
    )

    bg = b[ids].reshape(_E // 2, 2, _D)

    out = pl.pallas_call(
        _moe_body,
        grid_spec=grid_spec,
        out_shape=jax.ShapeDtypeStruct((_N, _D), jnp.float32),
        compiler_params=pltpu.CompilerParams(
            dimension_semantics=("arbitrary",),
        ),
    )(ids, m2d, x, W, W, bg)
    return out


# speedup vs baseline: 1.2051x; 1.0033x over previous
import jax
import jax.numpy as jnp
from jax.experimental import pallas as pl
from jax.experimental.pallas import tpu as pltpu

_E = 64
_D = 1024
_N = 128
_K = 4


def _moe_body(ids_ref, m_ref, x_ref, *rest):
    w_refs = rest[:_K]
    b_ref, o_ref = rest[_K], rest[_K + 1]
    i = pl.program_id(0)

    @pl.when(i == 0)
    def _init():
        o_ref[...] = jnp.zeros_like(o_ref)

    x = x_ref[...]
    m = m_ref[...]
    acc = jnp.zeros((_N, _D), jnp.float32)
    prev_e = None
    for j in range(_K):
        e = ids_ref[_K * i + j]
        xw = jax.lax.dot_general(
            x, w_refs[j][0], dimension_numbers=(((1,), (1,)), ((), ())),
            preferred_element_type=jnp.float32,
        ) + b_ref[0, j:j + 1, :]
        mask = m == e if j == 0 else (m == e) & (e != prev_e)
        acc = acc + jnp.where(mask, xw, 0.0)
        prev_e = e
    o_ref[...] += acc


def kernel(x, module_indices, W, b):
    m = module_indices.astype(jnp.int32)
    s = jnp.sort(m)
    is_new = jnp.concatenate([jnp.ones((1,), jnp.bool_), s[1:] != s[:-1]])
    pos = jnp.cumsum(is_new.astype(jnp.int32)) - 1
    ids = jnp.full((_E,), s[-1], jnp.int32).at[pos].set(s)
    n_used = pos[-1] + 1
    n_steps = (n_used + _K - 1) // _K

    m2d = m.reshape(_N, 1)
    bg = b[ids].reshape(_E // _K, _K, _D)

    def w_spec(j):
        return pl.BlockSpec(
            (1, _D, _D), lambda i, ids, j=j: (ids[_K * i + j], 0, 0))

    grid_spec = pltpu.PrefetchScalarGridSpec(
        num_scalar_prefetch=1,
        grid=(n_steps,),
        in_specs=[
            pl.BlockSpec((_N, 1), lambda i, ids: (0, 0)),
            pl.BlockSpec((_N, _D), lambda i, ids: (0, 0)),
            *[w_spec(j) for j in range(_K)],
            pl.BlockSpec((1, _K, _D), lambda i, ids: (i, 0, 0)),
        ],
        out_specs=pl.BlockSpec((_N, _D), lambda i, ids: (0, 0)),
    )

    out = pl.pallas_call(
        _moe_body,
        grid_spec=grid_spec,
        out_shape=jax.ShapeDtypeStruct((_N, _D), jnp.float32),
        compiler_params=pltpu.CompilerParams(
            dimension_semantics=("arbitrary",),
        ),
    )(ids, m2d, x, *([W] * _K), bg)
    return out


# speedup vs baseline: 1.2488x; 1.0363x over previous
import jax
import jax.numpy as jnp
from jax import lax
from jax.experimental import pallas as pl
from jax.experimental.pallas import tpu as pltpu

_E = 64
_D = 1024
_N = 128
_K = 4


def _route_body(m_ref, ids_ref, n_ref):
    m1x = m_ref[...]
    e_iota = lax.broadcasted_iota(jnp.int32, (_E, _N), 0)
    eq = (e_iota == m1x).astype(jnp.int32)
    pres = jnp.max(eq, axis=1, keepdims=True)
    row = lax.broadcasted_iota(jnp.int32, (_E, _E), 0)
    col = lax.broadcasted_iota(jnp.int32, (_E, _E), 1)
    lower = (row > col).astype(jnp.float32)
    xrank = lax.dot_general(
        lower, pres.astype(jnp.float32),
        dimension_numbers=(((1,), (0,)), ((), ())),
        preferred_element_type=jnp.float32,
    ).astype(jnp.int32)
    slot = (xrank == col) & (pres > 0)
    slot_i = slot.astype(jnp.int32)
    ids = jnp.sum(slot_i * row, axis=0, keepdims=True)
    filled = jnp.sum(slot_i, axis=0, keepdims=True)
    eids = lax.broadcasted_iota(jnp.int32, (_E, 1), 0)
    maxid = jnp.max(pres * eids)
    ids_ref[...] = ids + (1 - filled) * maxid
    n_ref[...] = jnp.sum(pres).reshape(1, 1)


def _route(m2d):
    return pl.pallas_call(
        _route_body,
        out_shape=[
            jax.ShapeDtypeStruct((1, _E), jnp.int32),
            jax.ShapeDtypeStruct((1, 1), jnp.int32),
        ],
    )(m2d)


def _moe_body(ids_ref, m_ref, x_ref, *rest):
    w_refs = rest[:_K]
    b_refs = rest[_K:2 * _K]
    o_ref = rest[2 * _K]
    i = pl.program_id(0)

    @pl.when(i == 0)
    def _init():
        o_ref[...] = jnp.zeros_like(o_ref)

    x = x_ref[...]
    m = m_ref[...]
    acc = jnp.zeros((_N, _D), jnp.float32)
    prev_e = None
    for j in range(_K):
        e = ids_ref[_K * i + j]
        xw = jax.lax.dot_general(
            x, w_refs[j][0], dimension_numbers=(((1,), (1,)), ((), ())),
            preferred_element_type=jnp.float32,
        ) + b_refs[j][0]
        mask = m == e if j == 0 else (m == e) & (e != prev_e)
        acc = acc + jnp.where(mask, xw, 0.0)
        prev_e = e
    o_ref[...] += acc


def kernel(x, module_indices, W, b):
    m = module_indices.astype(jnp.int32)
    ids2d, n2d = _route(m.reshape(1, _N))
    ids = ids2d.reshape(_E)
    n_used = n2d[0, 0]
    n_steps = (n_used + _K - 1) // _K

    m2d = m.reshape(_N, 1)
    b3 = b.reshape(_E, 1, _D)

    def w_spec(j):
        return pl.BlockSpec(
            (1, _D, _D), lambda i, ids, j=j: (ids[_K * i + j], 0, 0))

    def b_spec(j):
        return pl.BlockSpec(
            (1, 1, _D), lambda i, ids, j=j: (ids[_K * i + j], 0, 0))

    grid_spec = pltpu.PrefetchScalarGridSpec(
        num_scalar_prefetch=1,
        grid=(n_steps,),
        in_specs=[
            pl.BlockSpec((_N, 1), lambda i, ids: (0, 0)),
            pl.BlockSpec((_N, _D), lambda i, ids: (0, 0)),
            *[w_spec(j) for j in range(_K)],
            *[b_spec(j) for j in range(_K)],
        ],
        out_specs=pl.BlockSpec((_N, _D), lambda i, ids: (0, 0)),
    )

    out = pl.pallas_call(
        _moe_body,
        grid_spec=grid_spec,
        out_shape=jax.ShapeDtypeStruct((_N, _D), jnp.float32),
        compiler_params=pltpu.CompilerParams(
            dimension_semantics=("arbitrary",),
        ),
    )(ids, m2d, x, *([W] * _K), *([b3] * _K))
    return out


# speedup vs baseline: 1.3173x; 1.0548x over previous
import jax
import jax.numpy as jnp
from jax import lax
from jax.experimental import pallas as pl
from jax.experimental.pallas import tpu as pltpu

_E = 64
_D = 1024
_N = 128
_K = 4


def _route_body(m_ref, ids_ref, n_ref):
    m1x = m_ref[...]
    e_iota = lax.broadcasted_iota(jnp.int32, (_E, _N), 0)
    eq = (e_iota == m1x).astype(jnp.int32)
    pres = jnp.max(eq, axis=1, keepdims=True)
    row = lax.broadcasted_iota(jnp.int32, (_E, _E), 0)
    col = lax.broadcasted_iota(jnp.int32, (_E, _E), 1)
    lower = (row > col).astype(jnp.float32)
    xrank = lax.dot_general(
        lower, pres.astype(jnp.float32),
        dimension_numbers=(((1,), (0,)), ((), ())),
        preferred_element_type=jnp.float32,
    ).astype(jnp.int32)
    slot = (xrank == col) & (pres > 0)
    slot_i = slot.astype(jnp.int32)
    ids = jnp.sum(slot_i * row, axis=0, keepdims=True)
    filled = jnp.sum(slot_i, axis=0, keepdims=True)
    eids = lax.broadcasted_iota(jnp.int32, (_E, 1), 0)
    maxid = jnp.max(pres * eids)
    ids_ref[...] = ids + (1 - filled) * maxid
    n_ref[...] = jnp.sum(pres).reshape(1, 1)


def _route(m2d):
    return pl.pallas_call(
        _route_body,
        out_shape=[
            jax.ShapeDtypeStruct((1, _E), jnp.int32),
            jax.ShapeDtypeStruct((1, 1), jnp.int32),
        ],
    )(m2d)


def _moe_body(ids_ref, m_ref, x_ref, *rest):
    wt_refs = rest[:_K]
    wb_refs = rest[_K:2 * _K]
    b_refs = rest[2 * _K:3 * _K]
    o_ref = rest[3 * _K]
    i = pl.program_id(0)

    @pl.when(i == 0)
    def _init():
        o_ref[...] = jnp.zeros_like(o_ref)

    x = x_ref[...]
    m = m_ref[...]
    acc = jnp.zeros((_N, _D), jnp.float32)
    prev_e = None
    for j in range(_K):
        e = ids_ref[_K * i + j]
        xw_t = jax.lax.dot_general(
            x, wt_refs[j][0], dimension_numbers=(((1,), (1,)), ((), ())),
            preferred_element_type=jnp.float32,
        )
        xw_b = jax.lax.dot_general(
            x, wb_refs[j][0], dimension_numbers=(((1,), (1,)), ((), ())),
            preferred_element_type=jnp.float32,
        )
        xw = jnp.concatenate([xw_t, xw_b], axis=1) + b_refs[j][0]
        mask = m == e if j == 0 else (m == e) & (e != prev_e)
        acc = acc + jnp.where(mask, xw, 0.0)
        prev_e = e
    o_ref[...] += acc


def kernel(x, module_indices, W, b):
    m = module_indices.astype(jnp.int32)
    ids2d, n2d = _route(m.reshape(1, _N))
    ids = ids2d.reshape(_E)
    n_used = n2d[0, 0]
    n_steps = (n_used + _K - 1) // _K

    m2d = m.reshape(_N, 1)
    b3 = b.reshape(_E, 1, _D)

    def w_spec(j, h):
        return pl.BlockSpec(
            (1, _D // 2, _D), lambda i, ids, j=j, h=h: (ids[_K * i + j], h, 0))

    def b_spec(j):
        return pl.BlockSpec(
            (1, 1, _D), lambda i, ids, j=j: (ids[_K * i + j], 0, 0))

    grid_spec = pltpu.PrefetchScalarGridSpec(
        num_scalar_prefetch=1,
        grid=(n_steps,),
        in_specs=[
            pl.BlockSpec((_N, 1), lambda i, ids: (0, 0)),
            pl.BlockSpec((_N, _D), lambda i, ids: (0, 0)),
            *[w_spec(j, 0) for j in range(_K)],
            *[w_spec(j, 1) for j in range(_K)],
            *[b_spec(j) for j in range(_K)],
        ],
        out_specs=pl.BlockSpec((_N, _D), lambda i, ids: (0, 0)),
    )

    out = pl.pallas_call(
        _moe_body,
        grid_spec=grid_spec,
        out_shape=jax.ShapeDtypeStruct((_N, _D), jnp.float32),
        compiler_params=pltpu.CompilerParams(
            dimension_semantics=("arbitrary",),
        ),
    )(ids, m2d, x, *([W] * (2 * _K)), *([b3] * _K))
    return out
